# baseline (speedup 1.0000x reference)
import functools

import jax
import jax.numpy as jnp
from jax import lax
from jax.experimental import pallas as pl
from jax.experimental.pallas import tpu as pltpu
from jax.experimental.pallas import tpu_sc as plsc

N = 8192
C_DIM = 96
C_PAD = 128
D_DIM = 32
K_CODES = 8192
TN = 256
NPROG = N // TN


def _tc_body(x_ref, we_ref, be_ref, wd_ref, bd_ref, c_ref, idx_ref, dec_ref):
    h = jnp.dot(
        x_ref[...], we_ref[...], preferred_element_type=jnp.float32,
    ) + be_ref[...]
    c = c_ref[...]
    cross = -2.0 * lax.dot_general(
        h, c, (((1,), (1,)), ((), ())),
    )
    xt = jnp.sum(h * h, axis=1, keepdims=True)
    yt = jnp.sum(c * c, axis=1)
    d = (cross + xt) + yt[None, :]
    m = jnp.min(d, axis=1, keepdims=True)
    cols = lax.broadcasted_iota(jnp.int32, d.shape, 1)
    big = jnp.where(d <= m, cols, jnp.int32(2**30))
    idx_ref[...] = jnp.min(big, axis=1, keepdims=True)
    i = pl.program_id(0)
    cslice = c_ref[pl.ds(i * TN, TN), :]
    dec_ref[...] = jnp.dot(cslice, wd_ref[...]) + bd_ref[...]


def _tc_call(flatx, W_enc, b_enc2, W_dec, b_dec2, centroids):
    return pl.pallas_call(
        _tc_body,
        grid=(NPROG,),
        in_specs=[
            pl.BlockSpec((TN, C_DIM), lambda i: (i, 0)),
            pl.BlockSpec((C_DIM, D_DIM), lambda i: (0, 0)),
            pl.BlockSpec((1, D_DIM), lambda i: (0, 0)),
            pl.BlockSpec((D_DIM, C_PAD), lambda i: (0, 0)),
            pl.BlockSpec((1, C_PAD), lambda i: (0, 0)),
            pl.BlockSpec((K_CODES, D_DIM), lambda i: (0, 0)),
        ],
        out_specs=[
            pl.BlockSpec((TN, 1), lambda i: (i, 0)),
            pl.BlockSpec((TN, C_PAD), lambda i: (i, 0)),
        ],
        out_shape=[
            jax.ShapeDtypeStruct((N, 1), jnp.int32),
            jax.ShapeDtypeStruct((N, C_PAD), jnp.float32),
        ],
    )(flatx, W_enc, b_enc2, W_dec, b_dec2, centroids)


def _make_sc_gather():
    info = plsc.get_sparse_core_info()
    nw = info.num_cores * info.num_subcores
    b_per_w = N // nw
    chunk = 128
    nchunk = b_per_w // chunk
    mesh = plsc.VectorSubcoreMesh(core_axis_name="c", subcore_axis_name="s")

    @functools.partial(
        pl.kernel,
        mesh=mesh,
        out_type=jax.ShapeDtypeStruct((N, C_PAD), jnp.float32),
        scratch_types=[
            pltpu.VMEM((chunk,), jnp.int32),
            pltpu.VMEM((chunk, C_PAD), jnp.float32),
            pltpu.SemaphoreType.DMA,
        ],
    )
    def gather(table_hbm, idx_hbm, out_hbm, idx_v, rows_v, sem):
        wid = lax.axis_index("s") * info.num_cores + lax.axis_index("c")
        base = wid * b_per_w
        for j in range(nchunk):
            off = base + j * chunk
            pltpu.sync_copy(idx_hbm.at[pl.ds(off, chunk)], idx_v)
            pltpu.async_copy(table_hbm.at[idx_v], rows_v, sem).wait()
            pltpu.sync_copy(rows_v, out_hbm.at[pl.ds(off, chunk)])

    return gather


def kernel(x, W_enc, b_enc, W_dec, b_dec, centroids):
    flatx = x.reshape(-1, C_DIM)
    wd_pad = jnp.pad(W_dec, ((0, 0), (0, C_PAD - C_DIM)))
    bd_pad = jnp.pad(b_dec, (0, C_PAD - C_DIM)).reshape(1, C_PAD)
    idx2d, dec = _tc_call(
        flatx, W_enc, b_enc.reshape(1, D_DIM), wd_pad, bd_pad, centroids,
    )
    idx = idx2d.reshape(N)
    out = _make_sc_gather()(dec, idx)
    return out[:, :C_DIM].reshape(x.shape)
